# baseline (speedup 1.0000x reference)
import jax
import jax.numpy as jnp
from jax.experimental import pallas as pl
from jax.experimental.pallas import tpu as pltpu

_BLK = 128


def _gcn_fused_kernel(
    x_ref, pr_ref, pc_ref, w1_ref, b1_ref, w2_ref, b2_ref,
    out_ref, a_ref,
):
    i = pl.program_id(0)
    blk, n = pr_ref.shape
    row_g = jax.lax.broadcasted_iota(jnp.int32, (blk, n), 0) + i * blk
    col_g = jax.lax.broadcasted_iota(jnp.int32, (blk, n), 1)
    eye = jnp.where(row_g == col_g, jnp.float32(1.0), jnp.float32(0.0))
    a_blk = jax.nn.sigmoid(eye + 0.5 * (pr_ref[...] + pc_ref[...].T))
    a_ref[pl.ds(i * blk, blk), :] = a_blk

    @pl.when(i == pl.num_programs(0) - 1)
    def _():
        a = a_ref[...]
        dis_c = jax.lax.rsqrt(jnp.sum(a, axis=0, keepdims=True))
        dis_r = jax.lax.rsqrt(jnp.sum(a, axis=1, keepdims=True))
        a_hat = a * dis_c * dis_r
        xw = jnp.dot(x_ref[...], w1_ref[...], preferred_element_type=jnp.float32)
        h = jnp.maximum(
            jnp.dot(a_hat, xw, preferred_element_type=jnp.float32) + b1_ref[...],
            0.0,
        )
        hw = jnp.dot(h, w2_ref[...], preferred_element_type=jnp.float32)
        out_ref[...] = (
            jnp.dot(a_hat, hw, preferred_element_type=jnp.float32) + b2_ref[...]
        )


@jax.jit
def kernel(x, adaptive_params, W1, b1, W2, b2):
    n, din = x.shape
    hid = W1.shape[1]
    dout = W2.shape[1]
    k = n // _BLK
    return pl.pallas_call(
        _gcn_fused_kernel,
        grid=(k,),
        in_specs=[
            pl.BlockSpec((n, din), lambda i: (0, 0)),
            pl.BlockSpec((_BLK, n), lambda i: (i, 0)),
            pl.BlockSpec((n, _BLK), lambda i: (0, i)),
            pl.BlockSpec((din, hid), lambda i: (0, 0)),
            pl.BlockSpec((1, hid), lambda i: (0, 0)),
            pl.BlockSpec((hid, dout), lambda i: (0, 0)),
            pl.BlockSpec((1, dout), lambda i: (0, 0)),
        ],
        out_specs=pl.BlockSpec((n, dout), lambda i: (0, 0)),
        scratch_shapes=[
            pltpu.VMEM((n, n), jnp.float32),
        ],
        out_shape=jax.ShapeDtypeStruct((n, dout), x.dtype),
    )(
        x, adaptive_params, adaptive_params,
        W1, b1.reshape(1, -1), W2, b2.reshape(1, -1),
    )
